# baseline (speedup 1.0000x reference)
import jax
import jax.numpy as jnp
from jax.experimental import pallas as pl
from jax.experimental.pallas import tpu as pltpu

_KB = 2048


def _mm_kernel(x_ref, w_ref, b_ref, o_ref, *, k_total):
    k = pl.program_id(0)
    nk = pl.num_programs(0)

    @pl.when(k == 0)
    def _init():
        o_ref[...] = jnp.zeros_like(o_ref)

    @pl.when(k < nk - 1)
    def _full():
        x = x_ref[...].astype(jnp.bfloat16)
        w = w_ref[...].astype(jnp.bfloat16)
        o_ref[...] += jax.lax.dot(x, w, preferred_element_type=jnp.float32)

    @pl.when(k == nk - 1)
    def _tail():
        valid = k_total - (nk - 1) * _KB
        x = x_ref[...]
        w = w_ref[...]
        cols = jax.lax.broadcasted_iota(jnp.int32, x.shape, 1)
        rows = jax.lax.broadcasted_iota(jnp.int32, w.shape, 0)
        x = jnp.where(cols < valid, x, 0.0).astype(jnp.bfloat16)
        w = jnp.where(rows < valid, w, 0.0).astype(jnp.bfloat16)
        o_ref[...] += jax.lax.dot(x, w, preferred_element_type=jnp.float32)
        o_ref[...] += b_ref[...]


def kernel(X, W, b):
    B, K = X.shape
    _, N = W.shape
    nk = pl.cdiv(K, _KB)
    b2 = b.reshape(1, N)
    import functools

    return pl.pallas_call(
        functools.partial(_mm_kernel, k_total=K),
        grid=(nk,),
        in_specs=[
            pl.BlockSpec((B, _KB), lambda k: (0, k)),
            pl.BlockSpec((_KB, N), lambda k: (k, 0)),
            pl.BlockSpec((1, N), lambda k: (0, 0)),
        ],
        out_specs=pl.BlockSpec((B, N), lambda k: (0, 0)),
        out_shape=jax.ShapeDtypeStruct((B, N), jnp.float32),
        compiler_params=pltpu.CompilerParams(
            dimension_semantics=("arbitrary",),
        ),
    )(X, W, b2)


# speedup vs baseline: 1.0004x; 1.0004x over previous
import jax
import jax.numpy as jnp
from jax.experimental import pallas as pl
from jax.experimental.pallas import tpu as pltpu

_KB = 4096


def _mm_kernel(x_ref, w_ref, b_ref, o_ref, *, k_total):
    k = pl.program_id(0)
    nk = pl.num_programs(0)

    @pl.when(k == 0)
    def _init():
        o_ref[...] = jnp.zeros_like(o_ref)

    @pl.when(k < nk - 1)
    def _full():
        x = x_ref[...].astype(jnp.bfloat16)
        w = w_ref[...].astype(jnp.bfloat16)
        o_ref[...] += jax.lax.dot(x, w, preferred_element_type=jnp.float32)

    @pl.when(k == nk - 1)
    def _tail():
        valid = k_total - (nk - 1) * _KB
        x = x_ref[...]
        w = w_ref[...]
        cols = jax.lax.broadcasted_iota(jnp.int32, x.shape, 1)
        rows = jax.lax.broadcasted_iota(jnp.int32, w.shape, 0)
        x = jnp.where(cols < valid, x, 0.0).astype(jnp.bfloat16)
        w = jnp.where(rows < valid, w, 0.0).astype(jnp.bfloat16)
        o_ref[...] += jax.lax.dot(x, w, preferred_element_type=jnp.float32)
        o_ref[...] += b_ref[...]


def kernel(X, W, b):
    B, K = X.shape
    _, N = W.shape
    nk = pl.cdiv(K, _KB)
    b2 = b.reshape(1, N)
    import functools

    return pl.pallas_call(
        functools.partial(_mm_kernel, k_total=K),
        grid=(nk,),
        in_specs=[
            pl.BlockSpec((B, _KB), lambda k: (0, k)),
            pl.BlockSpec((_KB, N), lambda k: (k, 0)),
            pl.BlockSpec((1, N), lambda k: (0, 0)),
        ],
        out_specs=pl.BlockSpec((B, N), lambda k: (0, 0)),
        out_shape=jax.ShapeDtypeStruct((B, N), jnp.float32),
        compiler_params=pltpu.CompilerParams(
            dimension_semantics=("arbitrary",),
        ),
    )(X, W, b2)
